# Initial kernel scaffold; baseline (speedup 1.0000x reference)
import jax
import jax.numpy as jnp
from jax.experimental import pallas as pl


def kernel(edge_index, x, W_e0, b_e0, W_n0, b_n0, W_e1, b_e1, W_n1, b_n1, W_e2, b_e2, W_n2, b_n2, W_m0, b_m0, W_m1, b_m1, W_m2, b_m2):
    raise NotImplementedError("write your pallas kernel here")


# speedup vs baseline: 4.1169x; 4.1169x over previous
import functools
import jax
import jax.numpy as jnp
from jax import lax
from jax.experimental import pallas as pl
from jax.experimental.pallas import tpu as pltpu
from jax.experimental.pallas import tpu_sc as plsc

NC, NS = 2, 16
NW = NC * NS
CB = 128


def _make_sc_edge_kernel(E_pad, CH, Npad, DE, with_scatter):
  mesh = plsc.VectorSubcoreMesh(
      core_axis_name="c", subcore_axis_name="s",
      num_cores=NC, num_subcores=NS)

  out_type = [jax.ShapeDtypeStruct((E_pad, DE), jnp.float32)]
  if with_scatter:
    out_type += [
        jax.ShapeDtypeStruct((NC, Npad, DE), jnp.float32),
        jax.ShapeDtypeStruct((NC, Npad, 16), jnp.float32),
    ]

  scratch = [
      pltpu.VMEM((CH, CB), jnp.int32),
      pltpu.VMEM((CH, CB), jnp.int32),
      pltpu.VMEM((CB, DE), jnp.float32),
      pltpu.VMEM((CB, DE), jnp.float32),
      pltpu.VMEM((CB, DE), jnp.float32),
      pltpu.SemaphoreType.DMA,
      pltpu.SemaphoreType.DMA,
  ]
  if with_scatter:
    scratch += [
        pltpu.VMEM((CB, 16), jnp.float32),
        pltpu.VMEM_SHARED((Npad, DE), jnp.float32),
        pltpu.VMEM_SHARED((Npad, 16), jnp.float32),
    ]

  def body(*refs):
    if with_scatter:
      (src_hbm, dst_hbm, gs_hbm, gd_hbm, z32_hbm, z16_hbm, ones_hbm,
       out_hbm, sums_hbm, cnt_hbm,
       src_v, dst_v, gs_v, gd_v, eo_v, sem1, sem2,
       ones_v, sums_sh, cnt_sh) = refs
    else:
      (src_hbm, dst_hbm, gs_hbm, gd_hbm,
       out_hbm,
       src_v, dst_v, gs_v, gd_v, eo_v, sem1, sem2) = refs

    c = lax.axis_index("c")
    s = lax.axis_index("s")
    wid = c * NS + s

    if with_scatter:
      rp = Npad // NS
      pltpu.sync_copy(z32_hbm.at[pl.ds(s * rp, rp)],
                      sums_sh.at[pl.ds(s * rp, rp)])
      pltpu.sync_copy(z16_hbm.at[pl.ds(s * rp, rp)],
                      cnt_sh.at[pl.ds(s * rp, rp)])
      pltpu.sync_copy(ones_hbm, ones_v)
      plsc.subcore_barrier()

    pltpu.sync_copy(src_hbm.at[wid], src_v)
    pltpu.sync_copy(dst_hbm.at[wid], dst_v)

    def chunk(j, carry):
      base = (wid * CH + j) * CB
      cp1 = pltpu.async_copy(gs_hbm.at[src_v.at[j]], gs_v, sem1)
      cp2 = pltpu.async_copy(gd_hbm.at[dst_v.at[j]], gd_v, sem2)
      cp1.wait()
      cp2.wait()

      def row(r, carry2):
        for h in range(DE // 16):
          v = gs_v[r, pl.ds(h * 16, 16)] + gd_v[r, pl.ds(h * 16, 16)]
          if with_scatter:
            v = jnp.maximum(v, 0.0)
          eo_v[r, pl.ds(h * 16, 16)] = v
        return carry2

      lax.fori_loop(0, CB, row, 0, unroll=8)

      pltpu.sync_copy(eo_v, out_hbm.at[pl.ds(base, CB)])
      if with_scatter:
        pltpu.sync_copy(eo_v, sums_sh.at[dst_v.at[j]], add=True)
        pltpu.sync_copy(ones_v, cnt_sh.at[dst_v.at[j]], add=True)
      return carry

    lax.fori_loop(0, CH, chunk, 0)

    if with_scatter:
      plsc.subcore_barrier()

      @pl.when(s == 0)
      def _():
        pltpu.sync_copy(sums_sh, sums_hbm.at[c])
        pltpu.sync_copy(cnt_sh, cnt_hbm.at[c])

  return pl.kernel(
      body, out_type=out_type, mesh=mesh, scratch_types=scratch,
      compiler_params=pltpu.CompilerParams(use_tc_tiling_on_sc=False))


def _tables0_body(x_ref, ws_ref, wd_ref, b_ref, o1_ref, o2_ref):
  x = x_ref[...]
  o1_ref[...] = jnp.dot(x, ws_ref[...],
                        preferred_element_type=jnp.float32) + b_ref[...]
  o2_ref[...] = jnp.dot(x, wd_ref[...], preferred_element_type=jnp.float32)


def _agg_body(sums_ref, cnt_ref, x_ref, wnx_ref, wna_ref, bn_ref,
              w1s_ref, w1d_ref, be1_ref, o1_ref, o2_ref):
  sums = sums_ref[0] + sums_ref[1]
  cnt = cnt_ref[0, :, 0:1] + cnt_ref[1, :, 0:1]
  agg = sums / jnp.maximum(cnt, 1.0)
  n0 = jnp.dot(x_ref[...], wnx_ref[...], preferred_element_type=jnp.float32)
  n0 = n0 + jnp.dot(agg, wna_ref[...], preferred_element_type=jnp.float32)
  n0 = jnp.maximum(n0 + bn_ref[...], 0.0)
  o1_ref[...] = jnp.dot(n0, w1s_ref[...],
                        preferred_element_type=jnp.float32) + be1_ref[...]
  o2_ref[...] = jnp.dot(n0, w1d_ref[...], preferred_element_type=jnp.float32)


def _head_body(e0_ref, t_ref, w1e_ref, wm0_ref, bm0_ref, wm1_ref, bm1_ref,
               wm2_ref, bm2_ref, o_ref):
  e0 = e0_ref[...]
  e1 = jnp.maximum(
      t_ref[...] + jnp.dot(e0, w1e_ref[...],
                           preferred_element_type=jnp.float32), 0.0)
  ea = 0.5 * e0 + 1.5 * e1
  h = jnp.maximum(
      jnp.dot(ea, wm0_ref[...], preferred_element_type=jnp.float32)
      + bm0_ref[...], 0.0)
  h = jnp.maximum(
      jnp.dot(h, wm1_ref[...], preferred_element_type=jnp.float32)
      + bm1_ref[...], 0.0)
  o_ref[...] = jnp.dot(h, wm2_ref[...],
                       preferred_element_type=jnp.float32) + bm2_ref[...]


def kernel(edge_index, x, W_e0, b_e0, W_n0, b_n0, W_e1, b_e1, W_n1, b_n1,
           W_e2, b_e2, W_n2, b_n2, W_m0, b_m0, W_m1, b_m1, W_m2, b_m2):
  N, D = x.shape
  DE = W_e0.shape[1]
  E = edge_index.shape[1]

  CH = -(-E // (NW * CB))
  E_pad = NW * CB * CH
  Npad = ((N + 1 + NS * 8 - 1) // (NS * 8)) * (NS * 8)

  x = x.astype(jnp.float32)
  pad_e = E_pad - E
  src = jnp.concatenate(
      [edge_index[0].astype(jnp.int32), jnp.full((pad_e,), N, jnp.int32)]
  ).reshape(NW, CH, CB)
  dst = jnp.concatenate(
      [edge_index[1].astype(jnp.int32), jnp.full((pad_e,), N, jnp.int32)]
  ).reshape(NW, CH, CB)
  x_pad = jnp.pad(x, ((0, Npad - N), (0, 0)))

  z32 = jnp.zeros((Npad, DE), jnp.float32)
  z16 = jnp.zeros((Npad, 16), jnp.float32)
  ones128 = jnp.ones((CB, 16), jnp.float32)

  g0s, g0d = pl.pallas_call(
      _tables0_body,
      out_shape=[jax.ShapeDtypeStruct((Npad, DE), jnp.float32)] * 2,
  )(x_pad, W_e0[:D], W_e0[D:2 * D], b_e0.reshape(1, DE))

  sc_a = _make_sc_edge_kernel(E_pad, CH, Npad, DE, with_scatter=True)
  e0_full, sums2, cnt2 = sc_a(src, dst, g0s, g0d, z32, z16, ones128)

  g1s, g1d = pl.pallas_call(
      _agg_body,
      out_shape=[jax.ShapeDtypeStruct((Npad, DE), jnp.float32)] * 2,
  )(sums2, cnt2, x_pad, W_n0[:D], W_n0[D:], b_n0.reshape(1, D),
    W_e1[:D], W_e1[D:2 * D], b_e1.reshape(1, DE))

  sc_b = _make_sc_edge_kernel(E_pad, CH, Npad, DE, with_scatter=False)
  (t_full,) = sc_b(src, dst, g1s, g1d)

  BE = 2048
  grid = (E_pad // BE,)
  out = pl.pallas_call(
      _head_body,
      grid=grid,
      in_specs=[
          pl.BlockSpec((BE, DE), lambda i: (i, 0)),
          pl.BlockSpec((BE, DE), lambda i: (i, 0)),
          pl.BlockSpec((DE, DE), lambda i: (0, 0)),
          pl.BlockSpec((DE, DE), lambda i: (0, 0)),
          pl.BlockSpec((1, DE), lambda i: (0, 0)),
          pl.BlockSpec((DE, DE), lambda i: (0, 0)),
          pl.BlockSpec((1, DE), lambda i: (0, 0)),
          pl.BlockSpec((DE, 1), lambda i: (0, 0)),
          pl.BlockSpec((1, 1), lambda i: (0, 0)),
      ],
      out_specs=pl.BlockSpec((BE, 1), lambda i: (i, 0)),
      out_shape=jax.ShapeDtypeStruct((E_pad, 1), jnp.float32),
  )(e0_full, t_full, W_e1[2 * D:], W_m0, b_m0.reshape(1, DE),
    W_m1, b_m1.reshape(1, DE), W_m2, b_m2.reshape(1, 1))

  return out[:E]


# speedup vs baseline: 5.9201x; 1.4380x over previous
import functools
import jax
import jax.numpy as jnp
from jax import lax
from jax.experimental import pallas as pl
from jax.experimental.pallas import tpu as pltpu
from jax.experimental.pallas import tpu_sc as plsc

NC, NS = 2, 16
NW = NC * NS
CB = 128


def _make_sc_edge_kernel(E_pad, CH, Npad, DE, with_scatter):
  mesh = plsc.VectorSubcoreMesh(
      core_axis_name="c", subcore_axis_name="s",
      num_cores=NC, num_subcores=NS)

  P = 128 // DE
  QB = CB // P

  out_type = [jax.ShapeDtypeStruct((E_pad // P, P * DE), jnp.float32)]
  if with_scatter:
    out_type += [
        jax.ShapeDtypeStruct((NC, Npad, DE), jnp.float32),
        jax.ShapeDtypeStruct((NC, Npad, 16), jnp.float32),
    ]

  scratch = [
      pltpu.VMEM((CH, CB), jnp.int32),
      pltpu.VMEM((CH, CB), jnp.int32),
      pltpu.VMEM((CB, DE), jnp.float32),
      pltpu.VMEM((CB, DE), jnp.float32),
      pltpu.VMEM((QB, P * DE), jnp.float32),
      pltpu.SemaphoreType.DMA,
      pltpu.SemaphoreType.DMA,
  ]
  if with_scatter:
    scratch += [
        pltpu.VMEM((CB, DE), jnp.float32),
        pltpu.VMEM((CB, 16), jnp.float32),
        pltpu.VMEM_SHARED((Npad, DE), jnp.float32),
        pltpu.VMEM_SHARED((Npad, 16), jnp.float32),
    ]

  def body(*refs):
    if with_scatter:
      (src_hbm, dst_hbm, gs_hbm, gd_hbm, z32_hbm, z16_hbm, ones_hbm,
       out_hbm, sums_hbm, cnt_hbm,
       src_v, dst_v, gs_v, gd_v, ep_v, sem1, sem2,
       es_v, ones_v, sums_sh, cnt_sh) = refs
    else:
      (src_hbm, dst_hbm, gs_hbm, gd_hbm,
       out_hbm,
       src_v, dst_v, gs_v, gd_v, ep_v, sem1, sem2) = refs

    c = lax.axis_index("c")
    s = lax.axis_index("s")
    wid = c * NS + s

    if with_scatter:
      rp = Npad // NS
      pltpu.sync_copy(z32_hbm.at[pl.ds(s * rp, rp)],
                      sums_sh.at[pl.ds(s * rp, rp)])
      pltpu.sync_copy(z16_hbm.at[pl.ds(s * rp, rp)],
                      cnt_sh.at[pl.ds(s * rp, rp)])
      pltpu.sync_copy(ones_hbm, ones_v)
      plsc.subcore_barrier()

    pltpu.sync_copy(src_hbm.at[wid], src_v)
    pltpu.sync_copy(dst_hbm.at[wid], dst_v)

    def chunk(j, carry):
      qbase = (wid * CH + j) * QB
      cp1 = pltpu.async_copy(gs_hbm.at[src_v.at[j]], gs_v, sem1)
      cp2 = pltpu.async_copy(gd_hbm.at[dst_v.at[j]], gd_v, sem2)
      cp1.wait()
      cp2.wait()

      def row(q, carry2):
        for k in range(P):
          r = P * q + k
          for h in range(DE // 16):
            v = gs_v[r, pl.ds(h * 16, 16)] + gd_v[r, pl.ds(h * 16, 16)]
            if with_scatter:
              v = jnp.maximum(v, 0.0)
              es_v[r, pl.ds(h * 16, 16)] = v
            ep_v[q, pl.ds(k * DE + h * 16, 16)] = v
        return carry2

      lax.fori_loop(0, QB, row, 0, unroll=4)

      pltpu.sync_copy(ep_v, out_hbm.at[pl.ds(qbase, QB)])
      if with_scatter:
        pltpu.sync_copy(es_v, sums_sh.at[dst_v.at[j]], add=True)
        pltpu.sync_copy(ones_v, cnt_sh.at[dst_v.at[j]], add=True)
      return carry

    lax.fori_loop(0, CH, chunk, 0)

    if with_scatter:
      plsc.subcore_barrier()

      @pl.when(s == 0)
      def _():
        pltpu.sync_copy(sums_sh, sums_hbm.at[c])
        pltpu.sync_copy(cnt_sh, cnt_hbm.at[c])

  return pl.kernel(
      body, out_type=out_type, mesh=mesh, scratch_types=scratch,
      compiler_params=pltpu.CompilerParams(use_tc_tiling_on_sc=False))


def _tables0_body(x_ref, ws_ref, wd_ref, b_ref, o1_ref, o2_ref):
  x = x_ref[...]
  o1_ref[...] = jnp.dot(x, ws_ref[...],
                        preferred_element_type=jnp.float32) + b_ref[...]
  o2_ref[...] = jnp.dot(x, wd_ref[...], preferred_element_type=jnp.float32)


def _agg_body(sums_ref, cnt_ref, x_ref, wnx_ref, wna_ref, bn_ref,
              w1s_ref, w1d_ref, be1_ref, o1_ref, o2_ref):
  sums = sums_ref[0] + sums_ref[1]
  cnt = cnt_ref[0, :, 0:1] + cnt_ref[1, :, 0:1]
  agg = sums / jnp.maximum(cnt, 1.0)
  n0 = jnp.dot(x_ref[...], wnx_ref[...], preferred_element_type=jnp.float32)
  n0 = n0 + jnp.dot(agg, wna_ref[...], preferred_element_type=jnp.float32)
  n0 = jnp.maximum(n0 + bn_ref[...], 0.0)
  o1_ref[...] = jnp.dot(n0, w1s_ref[...],
                        preferred_element_type=jnp.float32) + be1_ref[...]
  o2_ref[...] = jnp.dot(n0, w1d_ref[...], preferred_element_type=jnp.float32)


def _head_body(e0_ref, t_ref, w1e_ref, wm0_ref, bm0_ref, wm1_ref, bm1_ref,
               wm2_ref, bm2_ref, o_ref):
  e0 = e0_ref[...]
  e1 = jnp.maximum(
      t_ref[...] + jnp.dot(e0, w1e_ref[...],
                           preferred_element_type=jnp.float32), 0.0)
  ea = 0.5 * e0 + 1.5 * e1
  h = jnp.maximum(
      jnp.dot(ea, wm0_ref[...], preferred_element_type=jnp.float32)
      + bm0_ref[...], 0.0)
  h = jnp.maximum(
      jnp.dot(h, wm1_ref[...], preferred_element_type=jnp.float32)
      + bm1_ref[...], 0.0)
  o_ref[...] = jnp.dot(h, wm2_ref[...],
                       preferred_element_type=jnp.float32) + bm2_ref[...]


def kernel(edge_index, x, W_e0, b_e0, W_n0, b_n0, W_e1, b_e1, W_n1, b_n1,
           W_e2, b_e2, W_n2, b_n2, W_m0, b_m0, W_m1, b_m1, W_m2, b_m2):
  N, D = x.shape
  DE = W_e0.shape[1]
  E = edge_index.shape[1]

  CH = -(-E // (NW * CB))
  E_pad = NW * CB * CH
  Npad = ((N + 1 + NS * 8 - 1) // (NS * 8)) * (NS * 8)

  x = x.astype(jnp.float32)
  pad_e = E_pad - E
  src = jnp.concatenate(
      [edge_index[0].astype(jnp.int32), jnp.full((pad_e,), N, jnp.int32)]
  ).reshape(NW, CH, CB)
  dst = jnp.concatenate(
      [edge_index[1].astype(jnp.int32), jnp.full((pad_e,), N, jnp.int32)]
  ).reshape(NW, CH, CB)
  x_pad = jnp.pad(x, ((0, Npad - N), (0, 0)))

  z32 = jnp.zeros((Npad, DE), jnp.float32)
  z16 = jnp.zeros((Npad, 16), jnp.float32)
  ones128 = jnp.ones((CB, 16), jnp.float32)

  g0s, g0d = pl.pallas_call(
      _tables0_body,
      out_shape=[jax.ShapeDtypeStruct((Npad, DE), jnp.float32)] * 2,
  )(x_pad, W_e0[:D], W_e0[D:2 * D], b_e0.reshape(1, DE))

  sc_a = _make_sc_edge_kernel(E_pad, CH, Npad, DE, with_scatter=True)
  e0_full, sums2, cnt2 = sc_a(src, dst, g0s, g0d, z32, z16, ones128)

  g1s, g1d = pl.pallas_call(
      _agg_body,
      out_shape=[jax.ShapeDtypeStruct((Npad, DE), jnp.float32)] * 2,
  )(sums2, cnt2, x_pad, W_n0[:D], W_n0[D:], b_n0.reshape(1, D),
    W_e1[:D], W_e1[D:2 * D], b_e1.reshape(1, DE))

  sc_b = _make_sc_edge_kernel(E_pad, CH, Npad, DE, with_scatter=False)
  (t_full,) = sc_b(src, dst, g1s, g1d)

  P = 128 // DE
  R = E_pad // P
  eye = jnp.eye(P, dtype=jnp.float32)
  w1e4 = jnp.kron(eye, W_e1[2 * D:])
  wm04 = jnp.kron(eye, W_m0)
  wm14 = jnp.kron(eye, W_m1)
  wm24 = jnp.kron(eye, W_m2)
  bm0t = jnp.tile(b_m0, P).reshape(1, P * DE)
  bm1t = jnp.tile(b_m1, P).reshape(1, P * DE)
  bm2t = jnp.tile(b_m2, P).reshape(1, P)

  BR = 1024
  grid = (R // BR,)
  out4 = pl.pallas_call(
      _head_body,
      grid=grid,
      in_specs=[
          pl.BlockSpec((BR, P * DE), lambda i: (i, 0)),
          pl.BlockSpec((BR, P * DE), lambda i: (i, 0)),
          pl.BlockSpec((P * DE, P * DE), lambda i: (0, 0)),
          pl.BlockSpec((P * DE, P * DE), lambda i: (0, 0)),
          pl.BlockSpec((1, P * DE), lambda i: (0, 0)),
          pl.BlockSpec((P * DE, P * DE), lambda i: (0, 0)),
          pl.BlockSpec((1, P * DE), lambda i: (0, 0)),
          pl.BlockSpec((P * DE, P), lambda i: (0, 0)),
          pl.BlockSpec((1, P), lambda i: (0, 0)),
      ],
      out_specs=pl.BlockSpec((BR, P), lambda i: (i, 0)),
      out_shape=jax.ShapeDtypeStruct((R, P), jnp.float32),
  )(e0_full, t_full, w1e4, wm04, bm0t, wm14, bm1t, wm24, bm2t)

  return out4.reshape(E_pad, 1)[:E]


# speedup vs baseline: 7.2792x; 1.2296x over previous
import functools
import jax
import jax.numpy as jnp
from jax import lax
from jax.experimental import pallas as pl
from jax.experimental.pallas import tpu as pltpu
from jax.experimental.pallas import tpu_sc as plsc

NC, NS = 2, 16
NW = NC * NS
CB = 128


def _make_sc_edge_kernel(E_pad, CH, Npad, DE, with_scatter):
  mesh = plsc.VectorSubcoreMesh(
      core_axis_name="c", subcore_axis_name="s",
      num_cores=NC, num_subcores=NS)

  P = 128 // DE
  QB = CB // P

  out_type = [jax.ShapeDtypeStruct((E_pad // P, P * DE), jnp.float32)]
  if with_scatter:
    out_type += [
        jax.ShapeDtypeStruct((NC, Npad, DE), jnp.float32),
        jax.ShapeDtypeStruct((NC, Npad, 16), jnp.float32),
    ]

  scratch = [
      pltpu.VMEM((CH, CB), jnp.int32),
      pltpu.VMEM((CH, CB), jnp.int32),
      pltpu.VMEM((CB, DE), jnp.float32),
      pltpu.VMEM((CB, DE), jnp.float32),
      pltpu.VMEM((CB, DE), jnp.float32),
      pltpu.VMEM((CB, DE), jnp.float32),
      pltpu.VMEM((QB, P * DE), jnp.float32),
      pltpu.SemaphoreType.DMA,
      pltpu.SemaphoreType.DMA,
  ]
  if with_scatter:
    scratch += [
        pltpu.VMEM((CB, DE), jnp.float32),
        pltpu.VMEM((CB, 16), jnp.float32),
        pltpu.VMEM_SHARED((Npad, DE), jnp.float32),
        pltpu.VMEM_SHARED((Npad, 16), jnp.float32),
    ]

  def body(*refs):
    if with_scatter:
      (src_hbm, dst_hbm, gs_hbm, gd_hbm, z32_hbm, z16_hbm, ones_hbm,
       out_hbm, sums_hbm, cnt_hbm,
       src_v, dst_v, gs_v0, gd_v0, gs_v1, gd_v1, ep_v, sem0, sem1,
       es_v, ones_v, sums_sh, cnt_sh) = refs
    else:
      (src_hbm, dst_hbm, gs_hbm, gd_hbm,
       out_hbm,
       src_v, dst_v, gs_v0, gd_v0, gs_v1, gd_v1, ep_v, sem0, sem1) = refs

    c = lax.axis_index("c")
    s = lax.axis_index("s")
    wid = c * NS + s

    if with_scatter:
      rp = Npad // NS
      pltpu.sync_copy(z32_hbm.at[pl.ds(s * rp, rp)],
                      sums_sh.at[pl.ds(s * rp, rp)])
      pltpu.sync_copy(z16_hbm.at[pl.ds(s * rp, rp)],
                      cnt_sh.at[pl.ds(s * rp, rp)])
      pltpu.sync_copy(ones_hbm, ones_v)
      plsc.subcore_barrier()

    pltpu.sync_copy(src_hbm.at[wid], src_v)
    pltpu.sync_copy(dst_hbm.at[wid], dst_v)

    def fire(j, gs_b, gd_b, sem):
      pltpu.async_copy(gs_hbm.at[src_v.at[j]], gs_b, sem)
      pltpu.async_copy(gd_hbm.at[dst_v.at[j]], gd_b, sem)

    def drain(gs_b, gd_b, sem):
      pltpu.make_async_copy(gs_hbm.at[pl.ds(0, CB)], gs_b, sem).wait()
      pltpu.make_async_copy(gs_hbm.at[pl.ds(0, CB)], gd_b, sem).wait()

    def process(j, gs_v, gd_v):
      qbase = (wid * CH + j) * QB

      def row(q, carry2):
        for k in range(P):
          r = P * q + k
          for h in range(DE // 16):
            v = gs_v[r, pl.ds(h * 16, 16)] + gd_v[r, pl.ds(h * 16, 16)]
            if with_scatter:
              v = jnp.maximum(v, 0.0)
              es_v[r, pl.ds(h * 16, 16)] = v
            ep_v[q, pl.ds(k * DE + h * 16, 16)] = v
        return carry2

      lax.fori_loop(0, QB, row, 0, unroll=4)

      pltpu.sync_copy(ep_v, out_hbm.at[pl.ds(qbase, QB)])
      if with_scatter:
        pltpu.sync_copy(es_v, sums_sh.at[dst_v.at[j]], add=True)
        pltpu.sync_copy(ones_v, cnt_sh.at[dst_v.at[j]], add=True)

    fire(0, gs_v0, gd_v0, sem0)
    fire(1, gs_v1, gd_v1, sem1)

    def pair(jj, carry):
      j0 = 2 * jj
      drain(gs_v0, gd_v0, sem0)
      process(j0, gs_v0, gd_v0)

      @pl.when(j0 + 2 < CH)
      def _():
        fire(j0 + 2, gs_v0, gd_v0, sem0)

      drain(gs_v1, gd_v1, sem1)
      process(j0 + 1, gs_v1, gd_v1)

      @pl.when(j0 + 3 < CH)
      def _():
        fire(j0 + 3, gs_v1, gd_v1, sem1)

      return carry

    lax.fori_loop(0, CH // 2, pair, 0)

    if with_scatter:
      plsc.subcore_barrier()

      @pl.when(s == 0)
      def _():
        pltpu.sync_copy(sums_sh, sums_hbm.at[c])
        pltpu.sync_copy(cnt_sh, cnt_hbm.at[c])

  return pl.kernel(
      body, out_type=out_type, mesh=mesh, scratch_types=scratch,
      compiler_params=pltpu.CompilerParams(use_tc_tiling_on_sc=False))


def _tables0_body(x_ref, ws_ref, wd_ref, b_ref, o1_ref, o2_ref):
  x = x_ref[...]
  o1_ref[...] = jnp.dot(x, ws_ref[...],
                        preferred_element_type=jnp.float32) + b_ref[...]
  o2_ref[...] = jnp.dot(x, wd_ref[...], preferred_element_type=jnp.float32)


def _agg_body(sums_ref, cnt_ref, x_ref, wnx_ref, wna_ref, bn_ref,
              w1s_ref, w1d_ref, be1_ref, o1_ref, o2_ref):
  sums = sums_ref[0] + sums_ref[1]
  cnt = cnt_ref[0, :, 0:1] + cnt_ref[1, :, 0:1]
  agg = sums / jnp.maximum(cnt, 1.0)
  n0 = jnp.dot(x_ref[...], wnx_ref[...], preferred_element_type=jnp.float32)
  n0 = n0 + jnp.dot(agg, wna_ref[...], preferred_element_type=jnp.float32)
  n0 = jnp.maximum(n0 + bn_ref[...], 0.0)
  o1_ref[...] = jnp.dot(n0, w1s_ref[...],
                        preferred_element_type=jnp.float32) + be1_ref[...]
  o2_ref[...] = jnp.dot(n0, w1d_ref[...], preferred_element_type=jnp.float32)


def _head_body(e0_ref, t_ref, w1e_ref, wm0_ref, bm0_ref, wm1_ref, bm1_ref,
               wm2_ref, bm2_ref, o_ref):
  e0 = e0_ref[...]
  e1 = jnp.maximum(
      t_ref[...] + jnp.dot(e0, w1e_ref[...],
                           preferred_element_type=jnp.float32), 0.0)
  ea = 0.5 * e0 + 1.5 * e1
  h = jnp.maximum(
      jnp.dot(ea, wm0_ref[...], preferred_element_type=jnp.float32)
      + bm0_ref[...], 0.0)
  h = jnp.maximum(
      jnp.dot(h, wm1_ref[...], preferred_element_type=jnp.float32)
      + bm1_ref[...], 0.0)
  o_ref[...] = jnp.dot(h, wm2_ref[...],
                       preferred_element_type=jnp.float32) + bm2_ref[...]


def kernel(edge_index, x, W_e0, b_e0, W_n0, b_n0, W_e1, b_e1, W_n1, b_n1,
           W_e2, b_e2, W_n2, b_n2, W_m0, b_m0, W_m1, b_m1, W_m2, b_m2):
  N, D = x.shape
  DE = W_e0.shape[1]
  E = edge_index.shape[1]

  CH = -(-E // (NW * CB))
  CH = CH + (CH % 2)
  E_pad = NW * CB * CH
  Npad = ((N + 1 + NS * 8 - 1) // (NS * 8)) * (NS * 8)

  x = x.astype(jnp.float32)
  pad_e = E_pad - E
  src = jnp.concatenate(
      [edge_index[0].astype(jnp.int32), jnp.full((pad_e,), N, jnp.int32)]
  ).reshape(NW, CH, CB)
  dst = jnp.concatenate(
      [edge_index[1].astype(jnp.int32), jnp.full((pad_e,), N, jnp.int32)]
  ).reshape(NW, CH, CB)
  x_pad = jnp.pad(x, ((0, Npad - N), (0, 0)))

  z32 = jnp.zeros((Npad, DE), jnp.float32)
  z16 = jnp.zeros((Npad, 16), jnp.float32)
  ones128 = jnp.ones((CB, 16), jnp.float32)

  g0s, g0d = pl.pallas_call(
      _tables0_body,
      out_shape=[jax.ShapeDtypeStruct((Npad, DE), jnp.float32)] * 2,
  )(x_pad, W_e0[:D], W_e0[D:2 * D], b_e0.reshape(1, DE))

  sc_a = _make_sc_edge_kernel(E_pad, CH, Npad, DE, with_scatter=True)
  e0_full, sums2, cnt2 = sc_a(src, dst, g0s, g0d, z32, z16, ones128)

  g1s, g1d = pl.pallas_call(
      _agg_body,
      out_shape=[jax.ShapeDtypeStruct((Npad, DE), jnp.float32)] * 2,
  )(sums2, cnt2, x_pad, W_n0[:D], W_n0[D:], b_n0.reshape(1, D),
    W_e1[:D], W_e1[D:2 * D], b_e1.reshape(1, DE))

  sc_b = _make_sc_edge_kernel(E_pad, CH, Npad, DE, with_scatter=False)
  (t_full,) = sc_b(src, dst, g1s, g1d)

  P = 128 // DE
  R = E_pad // P
  eye = jnp.eye(P, dtype=jnp.float32)
  w1e4 = jnp.kron(eye, W_e1[2 * D:])
  wm04 = jnp.kron(eye, W_m0)
  wm14 = jnp.kron(eye, W_m1)
  wm24 = jnp.kron(eye, W_m2)
  bm0t = jnp.tile(b_m0, P).reshape(1, P * DE)
  bm1t = jnp.tile(b_m1, P).reshape(1, P * DE)
  bm2t = jnp.tile(b_m2, P).reshape(1, P)

  BR = 1024
  grid = (R // BR,)
  out4 = pl.pallas_call(
      _head_body,
      grid=grid,
      in_specs=[
          pl.BlockSpec((BR, P * DE), lambda i: (i, 0)),
          pl.BlockSpec((BR, P * DE), lambda i: (i, 0)),
          pl.BlockSpec((P * DE, P * DE), lambda i: (0, 0)),
          pl.BlockSpec((P * DE, P * DE), lambda i: (0, 0)),
          pl.BlockSpec((1, P * DE), lambda i: (0, 0)),
          pl.BlockSpec((P * DE, P * DE), lambda i: (0, 0)),
          pl.BlockSpec((1, P * DE), lambda i: (0, 0)),
          pl.BlockSpec((P * DE, P), lambda i: (0, 0)),
          pl.BlockSpec((1, P), lambda i: (0, 0)),
      ],
      out_specs=pl.BlockSpec((BR, P), lambda i: (i, 0)),
      out_shape=jax.ShapeDtypeStruct((R, P), jnp.float32),
  )(e0_full, t_full, w1e4, wm04, bm0t, wm14, bm1t, wm24, bm2t)

  return out4.reshape(E_pad, 1)[:E]


# speedup vs baseline: 7.7145x; 1.0598x over previous
import functools
import jax
import jax.numpy as jnp
from jax import lax
from jax.experimental import pallas as pl
from jax.experimental.pallas import tpu as pltpu
from jax.experimental.pallas import tpu_sc as plsc

NC, NS = 2, 16
NW = NC * NS
CB = 128


def _make_sc_edge_kernel(E_pad, CH, Npad, DE, with_scatter):
  mesh = plsc.VectorSubcoreMesh(
      core_axis_name="c", subcore_axis_name="s",
      num_cores=NC, num_subcores=NS)

  P = 128 // DE
  QB = CB // P

  SW = DE + 16

  out_type = [jax.ShapeDtypeStruct((E_pad // P, P * DE), jnp.float32)]
  if with_scatter:
    out_type += [jax.ShapeDtypeStruct((NC, Npad, SW), jnp.float32)]

  scratch = [
      pltpu.VMEM((CH, CB), jnp.int32),
      pltpu.VMEM((CH, CB), jnp.int32),
      pltpu.VMEM((CB, DE), jnp.float32),
      pltpu.VMEM((CB, DE), jnp.float32),
      pltpu.VMEM((CB, DE), jnp.float32),
      pltpu.VMEM((CB, DE), jnp.float32),
      pltpu.VMEM((QB, P * DE), jnp.float32),
      pltpu.VMEM((QB, P * DE), jnp.float32),
      pltpu.SemaphoreType.DMA,
      pltpu.SemaphoreType.DMA,
      pltpu.SemaphoreType.DMA,
      pltpu.SemaphoreType.DMA,
  ]
  if with_scatter:
    scratch += [
        pltpu.VMEM((CB, SW), jnp.float32),
        pltpu.VMEM((CB, SW), jnp.float32),
        pltpu.VMEM_SHARED((Npad, SW), jnp.float32),
    ]

  def body(*refs):
    if with_scatter:
      (src_hbm, dst_hbm, gs_hbm, gd_hbm, zsw_hbm,
       out_hbm, sums_hbm,
       src_v, dst_v, gs_v0, gd_v0, gs_v1, gd_v1, ep_v0, ep_v1,
       semg0, semg1, semw0, semw1,
       es_v0, es_v1, acc_sh) = refs
      es_bufs = (es_v0, es_v1)
    else:
      (src_hbm, dst_hbm, gs_hbm, gd_hbm,
       out_hbm,
       src_v, dst_v, gs_v0, gd_v0, gs_v1, gd_v1, ep_v0, ep_v1,
       semg0, semg1, semw0, semw1) = refs
      es_bufs = (None, None)

    c = lax.axis_index("c")
    s = lax.axis_index("s")
    wid = c * NS + s

    if with_scatter:
      rp = Npad // NS
      pltpu.sync_copy(zsw_hbm.at[pl.ds(s * rp, rp)],
                      acc_sh.at[pl.ds(s * rp, rp)])

      def ones_row(r, carry2):
        one = jnp.full((16,), 1.0, jnp.float32)
        es_v0[r, pl.ds(DE, 16)] = one
        es_v1[r, pl.ds(DE, 16)] = one
        return carry2

      lax.fori_loop(0, CB, ones_row, 0, unroll=8)
      plsc.subcore_barrier()

    pltpu.sync_copy(src_hbm.at[wid], src_v)
    pltpu.sync_copy(dst_hbm.at[wid], dst_v)

    def fire(j, gs_b, gd_b, sem):
      pltpu.async_copy(gs_hbm.at[src_v.at[j]], gs_b, sem)
      pltpu.async_copy(gd_hbm.at[dst_v.at[j]], gd_b, sem)

    def drain_gather(gs_b, gd_b, sem):
      pltpu.make_async_copy(gs_hbm.at[pl.ds(0, CB)], gs_b, sem).wait()
      pltpu.make_async_copy(gs_hbm.at[pl.ds(0, CB)], gd_b, sem).wait()

    def drain_writes(ep_b, es_b, sem):
      pltpu.make_async_copy(ep_b, out_hbm.at[pl.ds(0, QB)], sem).wait()

    def process(j, gs_v, gd_v, ep_v, es_v, semw):
      qbase = (wid * CH + j) * QB

      def row(q, carry2):
        for k in range(P):
          r = P * q + k
          for h in range(DE // 16):
            v = gs_v[r, pl.ds(h * 16, 16)] + gd_v[r, pl.ds(h * 16, 16)]
            if with_scatter:
              v = jnp.maximum(v, 0.0)
              es_v[r, pl.ds(h * 16, 16)] = v
            ep_v[q, pl.ds(k * DE + h * 16, 16)] = v
        return carry2

      lax.fori_loop(0, QB, row, 0, unroll=4)

      pltpu.async_copy(ep_v, out_hbm.at[pl.ds(qbase, QB)], semw)
      if with_scatter:
        pltpu.sync_copy(es_v, acc_sh.at[dst_v.at[j]], add=True)

    fire(0, gs_v0, gd_v0, semg0)
    fire(1, gs_v1, gd_v1, semg1)

    def pair(jj, carry):
      j0 = 2 * jj
      drain_gather(gs_v0, gd_v0, semg0)

      @pl.when(j0 >= 2)
      def _():
        drain_writes(ep_v0, es_bufs[0], semw0)

      process(j0, gs_v0, gd_v0, ep_v0, es_bufs[0], semw0)

      @pl.when(j0 + 2 < CH)
      def _():
        fire(j0 + 2, gs_v0, gd_v0, semg0)

      drain_gather(gs_v1, gd_v1, semg1)

      @pl.when(j0 >= 2)
      def _():
        drain_writes(ep_v1, es_bufs[1], semw1)

      process(j0 + 1, gs_v1, gd_v1, ep_v1, es_bufs[1], semw1)

      @pl.when(j0 + 3 < CH)
      def _():
        fire(j0 + 3, gs_v1, gd_v1, semg1)

      return carry

    lax.fori_loop(0, CH // 2, pair, 0)
    drain_writes(ep_v0, es_bufs[0], semw0)
    drain_writes(ep_v1, es_bufs[1], semw1)

    if with_scatter:
      plsc.subcore_barrier()

      @pl.when(s == 0)
      def _():
        pltpu.sync_copy(acc_sh, sums_hbm.at[c])

  return pl.kernel(
      body, out_type=out_type, mesh=mesh, scratch_types=scratch,
      compiler_params=pltpu.CompilerParams(use_tc_tiling_on_sc=False))


def _tables0_body(x_ref, ws_ref, wd_ref, b_ref, o1_ref, o2_ref):
  x = x_ref[...]
  o1_ref[...] = jnp.dot(x, ws_ref[...],
                        preferred_element_type=jnp.float32) + b_ref[...]
  o2_ref[...] = jnp.dot(x, wd_ref[...], preferred_element_type=jnp.float32)


def _agg_body(acc_ref, x_ref, wnx_ref, wna_ref, bn_ref,
              w1s_ref, w1d_ref, be1_ref, o1_ref, o2_ref):
  DE = wna_ref.shape[0]
  acc = acc_ref[0] + acc_ref[1]
  sums = acc[:, :DE]
  cnt = acc[:, DE:DE + 1]
  agg = sums / jnp.maximum(cnt, 1.0)
  n0 = jnp.dot(x_ref[...], wnx_ref[...], preferred_element_type=jnp.float32)
  n0 = n0 + jnp.dot(agg, wna_ref[...], preferred_element_type=jnp.float32)
  n0 = jnp.maximum(n0 + bn_ref[...], 0.0)
  o1_ref[...] = jnp.dot(n0, w1s_ref[...],
                        preferred_element_type=jnp.float32) + be1_ref[...]
  o2_ref[...] = jnp.dot(n0, w1d_ref[...], preferred_element_type=jnp.float32)


def _head_body(e0_ref, t_ref, w1e_ref, wm0_ref, bm0_ref, wm1_ref, bm1_ref,
               wm2_ref, bm2_ref, o_ref):
  e0 = e0_ref[...]
  e1 = jnp.maximum(
      t_ref[...] + jnp.dot(e0, w1e_ref[...],
                           preferred_element_type=jnp.float32), 0.0)
  ea = 0.5 * e0 + 1.5 * e1
  h = jnp.maximum(
      jnp.dot(ea, wm0_ref[...], preferred_element_type=jnp.float32)
      + bm0_ref[...], 0.0)
  h = jnp.maximum(
      jnp.dot(h, wm1_ref[...], preferred_element_type=jnp.float32)
      + bm1_ref[...], 0.0)
  o_ref[...] = jnp.dot(h, wm2_ref[...],
                       preferred_element_type=jnp.float32) + bm2_ref[...]


def kernel(edge_index, x, W_e0, b_e0, W_n0, b_n0, W_e1, b_e1, W_n1, b_n1,
           W_e2, b_e2, W_n2, b_n2, W_m0, b_m0, W_m1, b_m1, W_m2, b_m2):
  N, D = x.shape
  DE = W_e0.shape[1]
  E = edge_index.shape[1]

  CH = -(-E // (NW * CB))
  CH = CH + (CH % 2)
  E_pad = NW * CB * CH
  Npad = ((N + 1 + NS * 8 - 1) // (NS * 8)) * (NS * 8)

  x = x.astype(jnp.float32)
  pad_e = E_pad - E
  src = jnp.concatenate(
      [edge_index[0].astype(jnp.int32), jnp.full((pad_e,), N, jnp.int32)]
  ).reshape(NW, CH, CB)
  dst = jnp.concatenate(
      [edge_index[1].astype(jnp.int32), jnp.full((pad_e,), N, jnp.int32)]
  ).reshape(NW, CH, CB)
  x_pad = jnp.pad(x, ((0, Npad - N), (0, 0)))

  zsw = jnp.zeros((Npad, DE + 16), jnp.float32)

  g0s, g0d = pl.pallas_call(
      _tables0_body,
      out_shape=[jax.ShapeDtypeStruct((Npad, DE), jnp.float32)] * 2,
  )(x_pad, W_e0[:D], W_e0[D:2 * D], b_e0.reshape(1, DE))

  sc_a = _make_sc_edge_kernel(E_pad, CH, Npad, DE, with_scatter=True)
  e0_full, acc2 = sc_a(src, dst, g0s, g0d, zsw)

  g1s, g1d = pl.pallas_call(
      _agg_body,
      out_shape=[jax.ShapeDtypeStruct((Npad, DE), jnp.float32)] * 2,
  )(acc2, x_pad, W_n0[:D], W_n0[D:], b_n0.reshape(1, D),
    W_e1[:D], W_e1[D:2 * D], b_e1.reshape(1, DE))

  sc_b = _make_sc_edge_kernel(E_pad, CH, Npad, DE, with_scatter=False)
  (t_full,) = sc_b(src, dst, g1s, g1d)

  P = 128 // DE
  R = E_pad // P
  eye = jnp.eye(P, dtype=jnp.float32)
  w1e4 = jnp.kron(eye, W_e1[2 * D:])
  wm04 = jnp.kron(eye, W_m0)
  wm14 = jnp.kron(eye, W_m1)
  wm24 = jnp.kron(eye, W_m2)
  bm0t = jnp.tile(b_m0, P).reshape(1, P * DE)
  bm1t = jnp.tile(b_m1, P).reshape(1, P * DE)
  bm2t = jnp.tile(b_m2, P).reshape(1, P)

  R_out = E // P
  BR = 1000
  assert R_out % BR == 0 and R_out <= R
  grid = (R_out // BR,)
  out4 = pl.pallas_call(
      _head_body,
      grid=grid,
      in_specs=[
          pl.BlockSpec((BR, P * DE), lambda i: (i, 0)),
          pl.BlockSpec((BR, P * DE), lambda i: (i, 0)),
          pl.BlockSpec((P * DE, P * DE), lambda i: (0, 0)),
          pl.BlockSpec((P * DE, P * DE), lambda i: (0, 0)),
          pl.BlockSpec((1, P * DE), lambda i: (0, 0)),
          pl.BlockSpec((P * DE, P * DE), lambda i: (0, 0)),
          pl.BlockSpec((1, P * DE), lambda i: (0, 0)),
          pl.BlockSpec((P * DE, P), lambda i: (0, 0)),
          pl.BlockSpec((1, P), lambda i: (0, 0)),
      ],
      out_specs=pl.BlockSpec((BR, P), lambda i: (i, 0)),
      out_shape=jax.ShapeDtypeStruct((R_out, P), jnp.float32),
  )(e0_full, t_full, w1e4, wm04, bm0t, wm14, bm1t, wm24, bm2t)

  return out4.reshape(E, 1)


# speedup vs baseline: 7.7653x; 1.0066x over previous
import functools
import jax
import jax.numpy as jnp
from jax import lax
from jax.experimental import pallas as pl
from jax.experimental.pallas import tpu as pltpu
from jax.experimental.pallas import tpu_sc as plsc

NC, NS = 2, 16
NW = NC * NS
CB = 128


def _make_sc_edge_kernel(E_pad, CH, Npad, DE, with_scatter):
  mesh = plsc.VectorSubcoreMesh(
      core_axis_name="c", subcore_axis_name="s",
      num_cores=NC, num_subcores=NS)

  P = 128 // DE
  QB = CB // P

  SW = DE + 16

  out_type = [jax.ShapeDtypeStruct((E_pad // P, P * DE), jnp.float32)]
  if with_scatter:
    out_type += [jax.ShapeDtypeStruct((NC, Npad, SW), jnp.float32)]

  scratch = [
      pltpu.VMEM((CH, CB), jnp.int32),
      pltpu.VMEM((CH, CB), jnp.int32),
      pltpu.VMEM((CB, DE), jnp.float32),
      pltpu.VMEM((CB, DE), jnp.float32),
      pltpu.VMEM((CB, DE), jnp.float32),
      pltpu.VMEM((CB, DE), jnp.float32),
      pltpu.VMEM((CB, DE), jnp.float32),
      pltpu.VMEM((CB, DE), jnp.float32),
      pltpu.VMEM((CB, DE), jnp.float32),
      pltpu.VMEM((CB, DE), jnp.float32),
      pltpu.VMEM((QB, P * DE), jnp.float32),
      pltpu.VMEM((QB, P * DE), jnp.float32),
      pltpu.SemaphoreType.DMA,
      pltpu.SemaphoreType.DMA,
      pltpu.SemaphoreType.DMA,
      pltpu.SemaphoreType.DMA,
      pltpu.SemaphoreType.DMA,
      pltpu.SemaphoreType.DMA,
  ]
  if with_scatter:
    scratch += [
        pltpu.VMEM((CB, SW), jnp.float32),
        pltpu.VMEM((CB, SW), jnp.float32),
        pltpu.VMEM_SHARED((Npad, SW), jnp.float32),
    ]

  def body(*refs):
    if with_scatter:
      (src_hbm, dst_hbm, gs_hbm, gd_hbm, zsw_hbm,
       out_hbm, sums_hbm,
       src_v, dst_v, gs_v0, gd_v0, gs_v1, gd_v1, gs_v2, gd_v2, gs_v3, gd_v3,
       ep_v0, ep_v1, semg0, semg1, semg2, semg3, semw0, semw1,
       es_v0, es_v1, acc_sh) = refs
      es_bufs = (es_v0, es_v1, es_v0, es_v1)
    else:
      (src_hbm, dst_hbm, gs_hbm, gd_hbm,
       out_hbm,
       src_v, dst_v, gs_v0, gd_v0, gs_v1, gd_v1, gs_v2, gd_v2, gs_v3, gd_v3,
       ep_v0, ep_v1, semg0, semg1, semg2, semg3, semw0, semw1) = refs
      es_bufs = (None, None, None, None)
    g_bufs = ((gs_v0, gd_v0, semg0), (gs_v1, gd_v1, semg1),
              (gs_v2, gd_v2, semg2), (gs_v3, gd_v3, semg3))
    ep_bufs = (ep_v0, ep_v1, ep_v0, ep_v1)
    w_sems = (semw0, semw1, semw0, semw1)

    c = lax.axis_index("c")
    s = lax.axis_index("s")
    wid = c * NS + s

    if with_scatter:
      rp = Npad // NS
      pltpu.sync_copy(zsw_hbm.at[pl.ds(s * rp, rp)],
                      acc_sh.at[pl.ds(s * rp, rp)])

      def ones_row(r, carry2):
        one = jnp.full((16,), 1.0, jnp.float32)
        es_v0[r, pl.ds(DE, 16)] = one
        es_v1[r, pl.ds(DE, 16)] = one
        return carry2

      lax.fori_loop(0, CB, ones_row, 0, unroll=8)
      plsc.subcore_barrier()

    pltpu.sync_copy(src_hbm.at[wid], src_v)
    pltpu.sync_copy(dst_hbm.at[wid], dst_v)

    def fire(j, gs_b, gd_b, sem):
      pltpu.async_copy(gs_hbm.at[src_v.at[j]], gs_b, sem)
      pltpu.async_copy(gd_hbm.at[dst_v.at[j]], gd_b, sem)

    def drain_gather(gs_b, gd_b, sem):
      pltpu.make_async_copy(gs_hbm.at[pl.ds(0, CB)], gs_b, sem).wait()
      pltpu.make_async_copy(gs_hbm.at[pl.ds(0, CB)], gd_b, sem).wait()

    def drain_writes(ep_b, es_b, sem):
      pltpu.make_async_copy(ep_b, out_hbm.at[pl.ds(0, QB)], sem).wait()

    def process(j, gs_v, gd_v, ep_v, es_v, semw):
      qbase = (wid * CH + j) * QB

      def row(q, carry2):
        for k in range(P):
          r = P * q + k
          for h in range(DE // 16):
            v = gs_v[r, pl.ds(h * 16, 16)] + gd_v[r, pl.ds(h * 16, 16)]
            if with_scatter:
              v = jnp.maximum(v, 0.0)
              es_v[r, pl.ds(h * 16, 16)] = v
            ep_v[q, pl.ds(k * DE + h * 16, 16)] = v
        return carry2

      lax.fori_loop(0, QB, row, 0, unroll=4)

      pltpu.async_copy(ep_v, out_hbm.at[pl.ds(qbase, QB)], semw)
      if with_scatter:
        pltpu.sync_copy(es_v, acc_sh.at[dst_v.at[j]], add=True)

    for b in range(4):
      fire(b, g_bufs[b][0], g_bufs[b][1], g_bufs[b][2])

    def quad(jj, carry):
      j0 = 4 * jj
      for b in range(4):
        j = j0 + b
        gs_b, gd_b, semg = g_bufs[b]
        drain_gather(gs_b, gd_b, semg)

        @pl.when(j >= 2)
        def _():
          drain_writes(ep_bufs[b], es_bufs[b], w_sems[b])

        process(j, gs_b, gd_b, ep_bufs[b], es_bufs[b], w_sems[b])

        @pl.when(j + 4 < CH)
        def _():
          fire(j + 4, gs_b, gd_b, semg)

      return carry

    lax.fori_loop(0, CH // 4, quad, 0)
    drain_writes(ep_v0, es_bufs[0], semw0)
    drain_writes(ep_v1, es_bufs[1], semw1)

    if with_scatter:
      plsc.subcore_barrier()

      @pl.when(s == 0)
      def _():
        pltpu.sync_copy(acc_sh, sums_hbm.at[c])

  return pl.kernel(
      body, out_type=out_type, mesh=mesh, scratch_types=scratch,
      compiler_params=pltpu.CompilerParams(use_tc_tiling_on_sc=False))


def _tables0_body(x_ref, ws_ref, wd_ref, b_ref, o1_ref, o2_ref):
  x = x_ref[...]
  o1_ref[...] = jnp.dot(x, ws_ref[...],
                        preferred_element_type=jnp.float32) + b_ref[...]
  o2_ref[...] = jnp.dot(x, wd_ref[...], preferred_element_type=jnp.float32)


def _agg_body(acc_ref, x_ref, wnx_ref, wna_ref, bn_ref,
              w1s_ref, w1d_ref, be1_ref, o1_ref, o2_ref):
  DE = wna_ref.shape[0]
  acc = acc_ref[0] + acc_ref[1]
  sums = acc[:, :DE]
  cnt = acc[:, DE:DE + 1]
  agg = sums / jnp.maximum(cnt, 1.0)
  n0 = jnp.dot(x_ref[...], wnx_ref[...], preferred_element_type=jnp.float32)
  n0 = n0 + jnp.dot(agg, wna_ref[...], preferred_element_type=jnp.float32)
  n0 = jnp.maximum(n0 + bn_ref[...], 0.0)
  o1_ref[...] = jnp.dot(n0, w1s_ref[...],
                        preferred_element_type=jnp.float32) + be1_ref[...]
  o2_ref[...] = jnp.dot(n0, w1d_ref[...], preferred_element_type=jnp.float32)


def _head_body(e0_ref, t_ref, w1e_ref, wm0_ref, bm0_ref, wm1_ref, bm1_ref,
               wm2_ref, bm2_ref, o_ref):
  e0 = e0_ref[...]
  e1 = jnp.maximum(
      t_ref[...] + jnp.dot(e0, w1e_ref[...],
                           preferred_element_type=jnp.float32), 0.0)
  ea = 0.5 * e0 + 1.5 * e1
  h = jnp.maximum(
      jnp.dot(ea, wm0_ref[...], preferred_element_type=jnp.float32)
      + bm0_ref[...], 0.0)
  h = jnp.maximum(
      jnp.dot(h, wm1_ref[...], preferred_element_type=jnp.float32)
      + bm1_ref[...], 0.0)
  o_ref[...] = jnp.dot(h, wm2_ref[...],
                       preferred_element_type=jnp.float32) + bm2_ref[...]


def kernel(edge_index, x, W_e0, b_e0, W_n0, b_n0, W_e1, b_e1, W_n1, b_n1,
           W_e2, b_e2, W_n2, b_n2, W_m0, b_m0, W_m1, b_m1, W_m2, b_m2):
  N, D = x.shape
  DE = W_e0.shape[1]
  E = edge_index.shape[1]

  CH = -(-E // (NW * CB))
  CH = ((CH + 3) // 4) * 4
  E_pad = NW * CB * CH
  Npad = ((N + 1 + NS * 8 - 1) // (NS * 8)) * (NS * 8)

  x = x.astype(jnp.float32)
  pad_e = E_pad - E
  src = jnp.concatenate(
      [edge_index[0].astype(jnp.int32), jnp.full((pad_e,), N, jnp.int32)]
  ).reshape(NW, CH, CB)
  dst = jnp.concatenate(
      [edge_index[1].astype(jnp.int32), jnp.full((pad_e,), N, jnp.int32)]
  ).reshape(NW, CH, CB)
  x_pad = jnp.pad(x, ((0, Npad - N), (0, 0)))

  zsw = jnp.zeros((Npad, DE + 16), jnp.float32)

  g0s, g0d = pl.pallas_call(
      _tables0_body,
      out_shape=[jax.ShapeDtypeStruct((Npad, DE), jnp.float32)] * 2,
  )(x_pad, W_e0[:D], W_e0[D:2 * D], b_e0.reshape(1, DE))

  sc_a = _make_sc_edge_kernel(E_pad, CH, Npad, DE, with_scatter=True)
  e0_full, acc2 = sc_a(src, dst, g0s, g0d, zsw)

  g1s, g1d = pl.pallas_call(
      _agg_body,
      out_shape=[jax.ShapeDtypeStruct((Npad, DE), jnp.float32)] * 2,
  )(acc2, x_pad, W_n0[:D], W_n0[D:], b_n0.reshape(1, D),
    W_e1[:D], W_e1[D:2 * D], b_e1.reshape(1, DE))

  sc_b = _make_sc_edge_kernel(E_pad, CH, Npad, DE, with_scatter=False)
  (t_full,) = sc_b(src, dst, g1s, g1d)

  P = 128 // DE
  R = E_pad // P
  eye = jnp.eye(P, dtype=jnp.float32)
  w1e4 = jnp.kron(eye, W_e1[2 * D:])
  wm04 = jnp.kron(eye, W_m0)
  wm14 = jnp.kron(eye, W_m1)
  wm24 = jnp.kron(eye, W_m2)
  bm0t = jnp.tile(b_m0, P).reshape(1, P * DE)
  bm1t = jnp.tile(b_m1, P).reshape(1, P * DE)
  bm2t = jnp.tile(b_m2, P).reshape(1, P)

  R_out = E // P
  BR = 1000
  assert R_out % BR == 0 and R_out <= R
  grid = (R_out // BR,)
  out4 = pl.pallas_call(
      _head_body,
      grid=grid,
      in_specs=[
          pl.BlockSpec((BR, P * DE), lambda i: (i, 0)),
          pl.BlockSpec((BR, P * DE), lambda i: (i, 0)),
          pl.BlockSpec((P * DE, P * DE), lambda i: (0, 0)),
          pl.BlockSpec((P * DE, P * DE), lambda i: (0, 0)),
          pl.BlockSpec((1, P * DE), lambda i: (0, 0)),
          pl.BlockSpec((P * DE, P * DE), lambda i: (0, 0)),
          pl.BlockSpec((1, P * DE), lambda i: (0, 0)),
          pl.BlockSpec((P * DE, P), lambda i: (0, 0)),
          pl.BlockSpec((1, P), lambda i: (0, 0)),
      ],
      out_specs=pl.BlockSpec((BR, P), lambda i: (i, 0)),
      out_shape=jax.ShapeDtypeStruct((R_out, P), jnp.float32),
  )(e0_full, t_full, w1e4, wm04, bm0t, wm14, bm1t, wm24, bm2t)

  return out4.reshape(E, 1)


# speedup vs baseline: 7.7871x; 1.0028x over previous
import jax
import jax.numpy as jnp
from jax import lax
from jax.experimental import pallas as pl
from jax.experimental.pallas import tpu as pltpu
from jax.experimental.pallas import tpu_sc as plsc

NC, NS = 2, 16
NW = NC * NS
CB = 128


def _make_sc_edge_kernel(E_pad, CH, Npad, DE, with_scatter):
  mesh = plsc.VectorSubcoreMesh(
      core_axis_name="c", subcore_axis_name="s",
      num_cores=NC, num_subcores=NS)

  P = 128 // DE
  QB = CB // P

  SW = DE + 16

  out_type = [jax.ShapeDtypeStruct((E_pad // P, P * DE), jnp.float32)]
  if with_scatter:
    out_type += [jax.ShapeDtypeStruct((NC, Npad, SW), jnp.float32)]

  scratch = [
      pltpu.VMEM((CH, CB), jnp.int32),
      pltpu.VMEM((CH, CB), jnp.int32),
      pltpu.VMEM((CB, DE), jnp.float32),
      pltpu.VMEM((CB, DE), jnp.float32),
      pltpu.VMEM((CB, DE), jnp.float32),
      pltpu.VMEM((CB, DE), jnp.float32),
      pltpu.VMEM((CB, DE), jnp.float32),
      pltpu.VMEM((CB, DE), jnp.float32),
      pltpu.VMEM((CB, DE), jnp.float32),
      pltpu.VMEM((CB, DE), jnp.float32),
      pltpu.VMEM((QB, P * DE), jnp.float32),
      pltpu.VMEM((QB, P * DE), jnp.float32),
      pltpu.SemaphoreType.DMA,
      pltpu.SemaphoreType.DMA,
      pltpu.SemaphoreType.DMA,
      pltpu.SemaphoreType.DMA,
      pltpu.SemaphoreType.DMA,
      pltpu.SemaphoreType.DMA,
  ]
  if with_scatter:
    scratch += [
        pltpu.VMEM((CB, SW), jnp.float32),
        pltpu.VMEM((CB, SW), jnp.float32),
        pltpu.VMEM_SHARED((Npad, SW), jnp.float32),
    ]

  def body(*refs):
    if with_scatter:
      (src_hbm, dst_hbm, gs_hbm, gd_hbm, zsw_hbm,
       out_hbm, sums_hbm,
       src_v, dst_v, gs_v0, gd_v0, gs_v1, gd_v1, gs_v2, gd_v2, gs_v3, gd_v3,
       ep_v0, ep_v1, semg0, semg1, semg2, semg3, semw0, semw1,
       es_v0, es_v1, acc_sh) = refs
      es_bufs = (es_v0, es_v1, es_v0, es_v1)
    else:
      (src_hbm, dst_hbm, gs_hbm, gd_hbm,
       out_hbm,
       src_v, dst_v, gs_v0, gd_v0, gs_v1, gd_v1, gs_v2, gd_v2, gs_v3, gd_v3,
       ep_v0, ep_v1, semg0, semg1, semg2, semg3, semw0, semw1) = refs
      es_bufs = (None, None, None, None)
    g_bufs = ((gs_v0, gd_v0, semg0), (gs_v1, gd_v1, semg1),
              (gs_v2, gd_v2, semg2), (gs_v3, gd_v3, semg3))
    ep_bufs = (ep_v0, ep_v1, ep_v0, ep_v1)
    w_sems = (semw0, semw1, semw0, semw1)

    c = lax.axis_index("c")
    s = lax.axis_index("s")
    wid = c * NS + s

    if with_scatter:
      rp = Npad // NS
      pltpu.sync_copy(zsw_hbm.at[pl.ds(s * rp, rp)],
                      acc_sh.at[pl.ds(s * rp, rp)])

      def ones_row(r, carry2):
        one = jnp.full((16,), 1.0, jnp.float32)
        es_v0[r, pl.ds(DE, 16)] = one
        es_v1[r, pl.ds(DE, 16)] = one
        return carry2

      lax.fori_loop(0, CB, ones_row, 0, unroll=8)
      plsc.subcore_barrier()

    pltpu.sync_copy(src_hbm.at[wid], src_v)
    pltpu.sync_copy(dst_hbm.at[wid], dst_v)

    def fire(j, gs_b, gd_b, sem):
      pltpu.async_copy(gs_hbm.at[src_v.at[j]], gs_b, sem)
      pltpu.async_copy(gd_hbm.at[dst_v.at[j]], gd_b, sem)

    def drain_gather(gs_b, gd_b, sem):
      pltpu.make_async_copy(gs_hbm.at[pl.ds(0, CB)], gs_b, sem).wait()
      pltpu.make_async_copy(gs_hbm.at[pl.ds(0, CB)], gd_b, sem).wait()

    def drain_writes(ep_b, es_b, sem):
      pltpu.make_async_copy(ep_b, out_hbm.at[pl.ds(0, QB)], sem).wait()

    def process(j, gs_v, gd_v, ep_v, es_v, semw):
      qbase = (wid * CH + j) * QB

      def row(q, carry2):
        for k in range(P):
          r = P * q + k
          for h in range(DE // 16):
            v = gs_v[r, pl.ds(h * 16, 16)] + gd_v[r, pl.ds(h * 16, 16)]
            if with_scatter:
              v = jnp.maximum(v, 0.0)
              es_v[r, pl.ds(h * 16, 16)] = v
            ep_v[q, pl.ds(k * DE + h * 16, 16)] = v
        return carry2

      lax.fori_loop(0, QB, row, 0, unroll=4)

      pltpu.async_copy(ep_v, out_hbm.at[pl.ds(qbase, QB)], semw)
      if with_scatter:
        pltpu.sync_copy(es_v, acc_sh.at[dst_v.at[j]], add=True)

    for b in range(4):
      fire(b, g_bufs[b][0], g_bufs[b][1], g_bufs[b][2])

    def quad(jj, carry):
      j0 = 4 * jj
      for b in range(4):
        j = j0 + b
        gs_b, gd_b, semg = g_bufs[b]
        drain_gather(gs_b, gd_b, semg)

        @pl.when(j >= 2)
        def _():
          drain_writes(ep_bufs[b], es_bufs[b], w_sems[b])

        process(j, gs_b, gd_b, ep_bufs[b], es_bufs[b], w_sems[b])

        @pl.when(j + 4 < CH)
        def _():
          fire(j + 4, gs_b, gd_b, semg)

      return carry

    lax.fori_loop(0, CH // 4, quad, 0)
    drain_writes(ep_v0, es_bufs[0], semw0)
    drain_writes(ep_v1, es_bufs[1], semw1)

    if with_scatter:
      plsc.subcore_barrier()

      @pl.when(s == 0)
      def _():
        pltpu.sync_copy(acc_sh, sums_hbm.at[c])

  return pl.kernel(
      body, out_type=out_type, mesh=mesh, scratch_types=scratch,
      compiler_params=pltpu.CompilerParams(use_tc_tiling_on_sc=False))


def _tables0_body(x_ref, ws_ref, wd_ref, b_ref, o1_ref, o2_ref):
  x = x_ref[...]
  o1_ref[...] = jnp.dot(x, ws_ref[...],
                        preferred_element_type=jnp.float32) + b_ref[...]
  o2_ref[...] = jnp.dot(x, wd_ref[...], preferred_element_type=jnp.float32)


def _agg_body(acc_ref, x_ref, wnx_ref, wna_ref, bn_ref,
              w1s_ref, w1d_ref, be1_ref, o1_ref, o2_ref):
  DE = wna_ref.shape[0]
  acc = acc_ref[0] + acc_ref[1]
  sums = acc[:, :DE]
  cnt = acc[:, DE:DE + 1]
  agg = sums / jnp.maximum(cnt, 1.0)
  n0 = jnp.dot(x_ref[...], wnx_ref[...], preferred_element_type=jnp.float32)
  n0 = n0 + jnp.dot(agg, wna_ref[...], preferred_element_type=jnp.float32)
  n0 = jnp.maximum(n0 + bn_ref[...], 0.0)
  o1_ref[...] = jnp.dot(n0, w1s_ref[...],
                        preferred_element_type=jnp.float32) + be1_ref[...]
  o2_ref[...] = jnp.dot(n0, w1d_ref[...], preferred_element_type=jnp.float32)


def _head_body(e0_ref, t_ref, w1e_ref, wm0_ref, bm0_ref, wm1_ref, bm1_ref,
               wm2_ref, bm2_ref, o_ref):
  e0 = e0_ref[...]
  e1 = jnp.maximum(
      t_ref[...] + jnp.dot(e0, w1e_ref[...],
                           preferred_element_type=jnp.float32), 0.0)
  ea = 0.5 * e0 + 1.5 * e1
  h = jnp.maximum(
      jnp.dot(ea, wm0_ref[...], preferred_element_type=jnp.float32)
      + bm0_ref[...], 0.0)
  h = jnp.maximum(
      jnp.dot(h, wm1_ref[...], preferred_element_type=jnp.float32)
      + bm1_ref[...], 0.0)
  o_ref[...] = jnp.dot(h, wm2_ref[...],
                       preferred_element_type=jnp.float32) + bm2_ref[...]


def kernel(edge_index, x, W_e0, b_e0, W_n0, b_n0, W_e1, b_e1, W_n1, b_n1,
           W_e2, b_e2, W_n2, b_n2, W_m0, b_m0, W_m1, b_m1, W_m2, b_m2):
  N, D = x.shape
  DE = W_e0.shape[1]
  E = edge_index.shape[1]

  CH = -(-E // (NW * CB))
  CH = ((CH + 3) // 4) * 4
  E_pad = NW * CB * CH
  Npad = ((N + 1 + NS * 8 - 1) // (NS * 8)) * (NS * 8)

  x = x.astype(jnp.float32)
  pad_e = E_pad - E
  src = jnp.concatenate(
      [edge_index[0].astype(jnp.int32), jnp.full((pad_e,), N, jnp.int32)]
  ).reshape(NW, CH, CB)
  dst = jnp.concatenate(
      [edge_index[1].astype(jnp.int32), jnp.full((pad_e,), N, jnp.int32)]
  ).reshape(NW, CH, CB)
  x_pad = jnp.pad(x, ((0, Npad - N), (0, 0)))

  zsw = jnp.zeros((Npad, DE + 16), jnp.float32)

  g0s, g0d = pl.pallas_call(
      _tables0_body,
      out_shape=[jax.ShapeDtypeStruct((Npad, DE), jnp.float32)] * 2,
  )(x_pad, W_e0[:D], W_e0[D:2 * D], b_e0.reshape(1, DE))

  sc_a = _make_sc_edge_kernel(E_pad, CH, Npad, DE, with_scatter=True)
  e0_full, acc2 = sc_a(src, dst, g0s, g0d, zsw)

  g1s, g1d = pl.pallas_call(
      _agg_body,
      out_shape=[jax.ShapeDtypeStruct((Npad, DE), jnp.float32)] * 2,
  )(acc2, x_pad, W_n0[:D], W_n0[D:], b_n0.reshape(1, D),
    W_e1[:D], W_e1[D:2 * D], b_e1.reshape(1, DE))

  sc_b = _make_sc_edge_kernel(E_pad, CH, Npad, DE, with_scatter=False)
  (t_full,) = sc_b(src, dst, g1s, g1d)

  P = 128 // DE
  R = E_pad // P
  eye = jnp.eye(P, dtype=jnp.float32)
  w1e4 = jnp.kron(eye, W_e1[2 * D:])
  wm04 = jnp.kron(eye, W_m0)
  wm14 = jnp.kron(eye, W_m1)
  wm24 = jnp.kron(eye, W_m2)
  bm0t = jnp.tile(b_m0, P).reshape(1, P * DE)
  bm1t = jnp.tile(b_m1, P).reshape(1, P * DE)
  bm2t = jnp.tile(b_m2, P).reshape(1, P)

  R_out = E // P
  BR = 1000
  assert R_out % BR == 0 and R_out <= R
  grid = (R_out // BR,)
  out4 = pl.pallas_call(
      _head_body,
      grid=grid,
      in_specs=[
          pl.BlockSpec((BR, P * DE), lambda i: (i, 0)),
          pl.BlockSpec((BR, P * DE), lambda i: (i, 0)),
          pl.BlockSpec((P * DE, P * DE), lambda i: (0, 0)),
          pl.BlockSpec((P * DE, P * DE), lambda i: (0, 0)),
          pl.BlockSpec((1, P * DE), lambda i: (0, 0)),
          pl.BlockSpec((P * DE, P * DE), lambda i: (0, 0)),
          pl.BlockSpec((1, P * DE), lambda i: (0, 0)),
          pl.BlockSpec((P * DE, P), lambda i: (0, 0)),
          pl.BlockSpec((1, P), lambda i: (0, 0)),
      ],
      out_specs=pl.BlockSpec((BR, P), lambda i: (i, 0)),
      out_shape=jax.ShapeDtypeStruct((R_out, P), jnp.float32),
  )(e0_full, t_full, w1e4, wm04, bm0t, wm14, bm1t, wm24, bm2t)

  return out4.reshape(E, 1)
